# baseline (speedup 1.0000x reference)
import functools

import jax
import jax.numpy as jnp
from jax import lax
from jax.experimental import pallas as pl
from jax.experimental.pallas import tpu as pltpu
import jax.experimental.pallas.tpu_sc as plsc

N_RADIAL = 5
N_ANGULAR = 8
RA = N_RADIAL * N_ANGULAR
TEMPLATE_RADIUS = 0.75
K_LRF = 16
ISC_DIMS = (32, 64, 128)
N_CLASSES = 40
B, N = 4, 1024
NT = 4
TIL = N // NT
CMAX = 128
GI = 8

_NC, _NS = 2, 16
_NW = _NC * _NS
_CH = 512

_HI = jax.lax.Precision.HIGHEST


def _dot(a, b):
  return jnp.dot(a, b, precision=_HI, preferred_element_type=jnp.float32)


def _dotd(a, b):
  return jnp.dot(a, b, preferred_element_type=jnp.float32)


def _rowmin_idx(x, iota, axis):
  m = jnp.min(x, axis=axis, keepdims=True)
  return jnp.min(jnp.where(x == m, iota, jnp.int32(1 << 30)), axis=axis)


def _geom_kernel(co_ref, nidx_ref, nbrs_ref):
  t = pl.program_id(1)
  co = co_ref[0]
  co_tile = co_ref[0, pl.ds(t * TIL, TIL), :]
  iota = lax.broadcasted_iota(jnp.int32, (TIL, N), 1)
  d2 = jnp.zeros((TIL, N), jnp.float32)
  for c in range(3):
    cc = jnp.reshape(co_tile[:, c], (TIL, 1))
    cr = jnp.reshape(co[:, c], (1, N))
    d2 = d2 + jnp.square(cc - cr)
  work = d2
  for j in range(K_LRF):
    amin = _rowmin_idx(work, iota, 1)
    sel = iota == amin[:, None]
    nbr = _dot(sel.astype(jnp.float32), co)
    nidx_ref[0, j:j + 1, :] = jnp.reshape(amin, (1, TIL))
    for c in range(3):
      nbrs_ref[0, j * 3 + c:j * 3 + c + 1, :] = jnp.reshape(nbr[:, c], (1, TIL))
    work = jnp.where(sel, jnp.float32(jnp.inf), work)


def _frames_kernel(put_ref, pvt_ref, tu_ref, tv_ref, m_ref):
  put = put_ref[0]
  pvt = pvt_ref[0]
  iota = lax.broadcasted_iota(jnp.int32, (K_LRF, N), 0)

  def body(ra, _):
    tu = jnp.reshape(tu_ref[ra], (1, 1))
    tv = jnp.reshape(tv_ref[ra], (1, 1))
    d2t = jnp.square(tu - put) + jnp.square(tv - pvt)
    work = d2t
    sels = []
    ws = []
    for j in range(3):
      amin = _rowmin_idx(work, iota, 0)
      sel = iota == amin[None, :]
      dmin = jnp.min(work, axis=0, keepdims=True)
      d3 = jnp.sqrt(jnp.maximum(dmin, 0.0) + jnp.float32(1e-8))
      ws.append(1.0 / (d3 + jnp.float32(1e-8)))
      sels.append(sel)
      work = jnp.where(sel, jnp.float32(jnp.inf), work)
    wsum = ws[0] + ws[1] + ws[2]
    m = jnp.zeros((K_LRF, N), jnp.float32)
    for j in range(3):
      m = m + sels[j].astype(jnp.float32) * (ws[j] / wsum)
    m_ref[0, ra] = m
    return 0

  lax.fori_loop(0, RA, body, 0)


def _sc_gather(table, idx):
  rows, d = idx.shape[0], table.shape[1]
  per_w = rows // _NW
  nch = per_w // _CH
  mesh = plsc.VectorSubcoreMesh(core_axis_name="c", subcore_axis_name="s")

  @functools.partial(
      pl.kernel, mesh=mesh,
      out_type=jax.ShapeDtypeStruct((rows, d), jnp.float32),
      scratch_types=[
          pltpu.VMEM((_CH,), jnp.int32),
          pltpu.VMEM((_CH, d), jnp.float32),
          pltpu.SemaphoreType.DMA,
      ],
  )
  def k(table_hbm, idx_hbm, out_hbm, idx_v, rows_v, sem):
    wid = lax.axis_index("s") * _NC + lax.axis_index("c")
    for s in range(nch):
      base = wid * per_w + s * _CH
      pltpu.sync_copy(idx_hbm.at[pl.ds(base, _CH)], idx_v)
      pltpu.async_copy(table_hbm.at[idx_v], rows_v, sem).wait()
      pltpu.sync_copy(rows_v, out_hbm.at[pl.ds(base, _CH)])

  return k(table, idx)


def _elu(x):
  return jnp.where(x > 0, x, jnp.exp(jnp.minimum(x, 0.0)) - 1.0)


def _mk_conv(cin, t, skip):
  def body(*refs):
    if skip:
      nb_ref, m5_ref, wf_ref, b_ref, x_ref, ws_ref, out_ref, acc_ref = refs
    else:
      nb_ref, m5_ref, wf_ref, b_ref, out_ref, acc_ref = refs
    acc_ref[...] = jnp.zeros((N, CMAX), jnp.float32)

    def gbody(g, _):
      mg = m5_ref[0, g]
      for j in range(GI):
        mgt = mg[:, j * K_LRF:(j + 1) * K_LRF]
        it = mgt[:, 0:1] * nb_ref[0, 0, :, :cin]
        for k in range(1, K_LRF):
          it = it + mgt[:, k:k + 1] * nb_ref[0, k, :, :cin]
        acc_ref[:, :t] += _dotd(it, wf_ref[g * GI + j])
      return 0

    lax.fori_loop(0, RA // GI, gbody, 0)
    h = acc_ref[:, :t] + b_ref[...]
    if skip:
      h = h + _dotd(x_ref[0], ws_ref[...])
    out_ref[0] = _elu(h)
  return body


def _conv_call(nbdim, cin, t, skip, args):
  bmap3 = lambda b: (b, 0, 0)
  bmap4 = lambda b: (b, 0, 0, 0)
  zmap2 = lambda b: (0, 0)
  zmap3 = lambda b: (0, 0, 0)
  in_specs = [
      pl.BlockSpec((1, K_LRF, N, nbdim), bmap4),
      pl.BlockSpec((1, RA // GI, N, GI * K_LRF), bmap4),
      pl.BlockSpec(args[2].shape, zmap3),
      pl.BlockSpec(args[3].shape, zmap2),
  ]
  if skip:
    in_specs += [pl.BlockSpec((1, N, args[4].shape[-1]), bmap3),
                 pl.BlockSpec(args[5].shape, zmap2)]
  return pl.pallas_call(
      _mk_conv(cin, t, skip),
      grid=(B,),
      in_specs=in_specs,
      out_specs=pl.BlockSpec((1, N, t), bmap3),
      out_shape=jax.ShapeDtypeStruct((B, N, t), jnp.float32),
      scratch_shapes=[pltpu.VMEM((N, CMAX), jnp.float32)],
  )(*args)


def _pool_kernel(x_ref, covp_ref):
  x = x_ref[0]
  mu = jnp.mean(x, axis=0, keepdims=True)
  xc = x - mu
  covp = lax.dot_general(xc, xc, (((0,), (0,)), ((), ())),
                         preferred_element_type=jnp.float32)
  covp_ref[0] = covp / jnp.float32(N)


def _clf_kernel(flat_ref, wc_ref, bc_ref, out_ref):
  out_ref[...] = _dotd(flat_ref[...], wc_ref[...]) + bc_ref[0:1, :]


@jax.jit
def kernel(inputs, params):
  f32 = jnp.float32
  bt3 = lambda b, t: (b, 0, t)

  co = inputs - jnp.mean(inputs, axis=1, keepdims=True)

  nidx, nbrs_t = pl.pallas_call(
      _geom_kernel,
      grid=(B, NT),
      in_specs=[pl.BlockSpec((1, N, 3), lambda b, t: (b, 0, 0))],
      out_specs=[
          pl.BlockSpec((1, K_LRF, TIL), bt3),
          pl.BlockSpec((1, 3 * K_LRF, TIL), bt3),
      ],
      out_shape=[
          jax.ShapeDtypeStruct((B, K_LRF, N), jnp.int32),
          jax.ShapeDtypeStruct((B, 3 * K_LRF, N), f32),
      ],
  )(co)

  nbrs = jnp.transpose(nbrs_t.reshape(B, K_LRF, 3, N), (0, 3, 1, 2))
  mu = jnp.mean(nbrs, axis=2, keepdims=True)
  cen = nbrs - mu
  cov = jnp.einsum('bnki,bnkj->bnij', cen, cen) / K_LRF
  _, v = jnp.linalg.eigh(cov)
  normal = v[..., 0]
  t1 = v[..., 2]
  t2 = v[..., 1]
  sgn = jnp.sign(jnp.sum(normal * co, axis=-1, keepdims=True) + 1e-9)
  signal = normal * sgn
  diff = nbrs - co[:, :, None, :]
  pu = jnp.einsum('bnkc,bnc->bnk', diff, t1)
  pv = jnp.einsum('bnkc,bnc->bnk', diff, t2)
  put = jnp.transpose(pu, (0, 2, 1))
  pvt = jnp.transpose(pv, (0, 2, 1))

  radii = TEMPLATE_RADIUS * (
      jnp.arange(1, N_RADIAL + 1, dtype=f32) / N_RADIAL)
  ang = 2.0 * jnp.pi * jnp.arange(N_ANGULAR, dtype=f32) / N_ANGULAR
  tu = (radii[:, None] * jnp.cos(ang)[None, :]).reshape(RA, 1)
  tv = (radii[:, None] * jnp.sin(ang)[None, :]).reshape(RA, 1)

  bmap3 = lambda b: (b, 0, 0)
  bmap4 = lambda b: (b, 0, 0, 0)
  zmap2 = lambda b: (0, 0)

  m = pl.pallas_call(
      _frames_kernel,
      grid=(B,),
      in_specs=[
          pl.BlockSpec((1, K_LRF, N), bmap3),
          pl.BlockSpec((1, K_LRF, N), bmap3),
          pl.BlockSpec((RA, 1), zmap2),
          pl.BlockSpec((RA, 1), zmap2),
      ],
      out_specs=pl.BlockSpec((1, RA, K_LRF, N), bmap4),
      out_shape=jax.ShapeDtypeStruct((B, RA, K_LRF, N), f32),
  )(put, pvt, tu, tv)

  m5 = jnp.transpose(
      jnp.transpose(m, (0, 1, 3, 2)).reshape(B, RA // GI, GI, N, K_LRF),
      (0, 1, 3, 2, 4)).reshape(B, RA // GI, N, GI * K_LRF)

  gidx = (nidx + (jnp.arange(B, dtype=jnp.int32) * N)[:, None, None]
          ).reshape(-1)

  wf = {}
  cin = 3
  for i, dim in enumerate(ISC_DIMS):
    wf[(i, 1)] = jnp.transpose(params['W1_%d' % i], (1, 2, 3, 0)
                               ).reshape(RA, cin, dim)
    wf[(i, 2)] = jnp.transpose(params['W2_%d' % i], (1, 2, 3, 0)
                               ).reshape(RA, dim, dim)
    cin = dim

  x = signal
  cin = 3
  for i, dim in enumerate(ISC_DIMS):
    tbl = x.reshape(B * N, cin)
    if cin < CMAX:
      tbl = jnp.pad(tbl, ((0, 0), (0, CMAX - cin)))
    nb = _sc_gather(tbl, gidx).reshape(B, K_LRF, N, CMAX)
    h1 = _conv_call(CMAX, cin, dim, False,
                    (nb, m5, wf[(i, 1)], params['b1_%d' % i].reshape(1, dim)))
    tbl2 = h1.reshape(B * N, dim)
    if dim < CMAX:
      tbl2 = jnp.pad(tbl2, ((0, 0), (0, CMAX - dim)))
    nb2 = _sc_gather(tbl2, gidx).reshape(B, K_LRF, N, CMAX)
    x = _conv_call(CMAX, dim, dim, True,
                   (nb2, m5, wf[(i, 2)], params['b2_%d' % i].reshape(1, dim),
                    x, params['Ws_%d' % i]))
    cin = dim

  covp = pl.pallas_call(
      _pool_kernel,
      grid=(B,),
      in_specs=[pl.BlockSpec((1, N, CMAX), bmap3)],
      out_specs=pl.BlockSpec((1, CMAX, CMAX), bmap3),
      out_shape=jax.ShapeDtypeStruct((B, CMAX, CMAX), f32),
  )(x)

  flat = covp.reshape(B, CMAX * CMAX)
  out = pl.pallas_call(
      _clf_kernel,
      in_specs=[
          pl.BlockSpec(flat.shape, None),
          pl.BlockSpec(params['Wc'].shape, None),
          pl.BlockSpec((1, N_CLASSES), None),
      ],
      out_shape=jax.ShapeDtypeStruct((B, N_CLASSES), f32),
  )(flat, params['Wc'], params['bc'].reshape(1, N_CLASSES))
  return out
